# baseline (speedup 1.0000x reference)
import functools

import jax
import jax.numpy as jnp
from jax import lax
from jax.experimental import pallas as pl
from jax.experimental.pallas import tpu as pltpu
from jax.experimental.pallas import tpu_sc as plsc

NC = 2
NS = 16
C = 125
CW = 8
NBUF = 5


def _sc_accumulate(ei4, eu3, item2v, u_pad):
    k_chunks = eu3.shape[1]
    dh = item2v.shape[1]
    rows_per_tile = u_pad // NS
    zchunks = rows_per_tile // C

    zeros_h = jnp.zeros((C, dh), jnp.float32)
    zeros_c = jnp.zeros((C, CW), jnp.float32)
    ones_c = jnp.ones((C, CW), jnp.float32)

    mesh = plsc.VectorSubcoreMesh(core_axis_name="c", subcore_axis_name="s")

    @functools.partial(
        pl.kernel,
        mesh=mesh,
        compiler_params=pltpu.CompilerParams(use_tc_tiling_on_sc=False),
        out_type=[
            jax.ShapeDtypeStruct((NC, u_pad, dh), jnp.float32),
            jax.ShapeDtypeStruct((NC, u_pad, CW), jnp.float32),
        ],
        scratch_types=[
            pltpu.VMEM((k_chunks, C), jnp.int32),
            pltpu.VMEM((k_chunks, C), jnp.int32),
            [pltpu.VMEM((C, dh), jnp.float32)] * NBUF,
            pltpu.VMEM((C, CW), jnp.float32),
            pltpu.VMEM((C, CW), jnp.float32),
            pltpu.VMEM_SHARED((u_pad, dh), jnp.float32),
            pltpu.VMEM_SHARED((u_pad, CW), jnp.float32),
            [pltpu.SemaphoreType.DMA] * NBUF,
            [pltpu.SemaphoreType.DMA] * NBUF,
            pltpu.SemaphoreType.DMA,
        ],
    )
    def acc(ei_hbm, eu_hbm, item_hbm, z_hbm, zc_hbm, o_hbm, psums_hbm,
            pcnts_hbm, ei_v, eu_v, rows, zbuf8, ones_v, sums_sh,
            cnts_sh, gsem, ssem, csem):
        c = lax.axis_index("c")
        s = lax.axis_index("s")

        pltpu.sync_copy(z_hbm, rows[0])
        pltpu.sync_copy(zc_hbm, zbuf8)
        pltpu.sync_copy(o_hbm, ones_v)
        base = s * rows_per_tile
        for k in range(zchunks):
            pltpu.sync_copy(rows[0], sums_sh.at[pl.ds(base + k * C, C)])
            pltpu.sync_copy(zbuf8, cnts_sh.at[pl.ds(base + k * C, C)])
        pltpu.sync_copy(ei_hbm.at[c, s], ei_v)
        pltpu.sync_copy(eu_hbm.at[s], eu_v)
        plsc.subcore_barrier()

        def chunk_body(t, carry):
            j0 = t * NBUF
            gd = [pltpu.async_copy(item_hbm.at[ei_v.at[j0 + b]], rows[b],
                                   gsem[b])
                  for b in range(NBUF)]
            sd, cd = [], []
            for b in range(NBUF):
                gd[b].wait()
                sd.append(pltpu.async_copy(rows[b],
                                           sums_sh.at[eu_v.at[j0 + b]],
                                           ssem[b], add=True))
                cd.append(pltpu.async_copy(ones_v,
                                           cnts_sh.at[eu_v.at[j0 + b]],
                                           csem, add=True))
            for b in range(NBUF):
                sd[b].wait()
                cd[b].wait()
            return carry

        lax.fori_loop(0, k_chunks // NBUF, chunk_body, 0)
        plsc.subcore_barrier()

        pltpu.sync_copy(sums_sh.at[pl.ds(base, rows_per_tile)],
                        psums_hbm.at[c, pl.ds(base, rows_per_tile)])
        pltpu.sync_copy(cnts_sh.at[pl.ds(base, rows_per_tile)],
                        pcnts_hbm.at[c, pl.ds(base, rows_per_tile)])

    return acc(ei4, eu3, item2v, zeros_h, zeros_c, ones_c)


def _combine_kernel(p0, p1, c0, c1, out):
    denom = jnp.maximum((c0[:, 0:1] + c1[:, 0:1]) * 0.5, 1.0)
    out[:] = jnp.concatenate([p0[:], p1[:]], axis=1) / denom


def kernel(user_embedding, item_embedding, edge_user, edge_item):
    num_user, d = user_embedding.shape
    num_item = item_embedding.shape[0]
    dh = d // 2
    e = edge_user.shape[0]

    per_w = -(-e // NS)
    k_chunks = -(-per_w // C)
    k_chunks = -(-k_chunks // NBUF) * NBUF
    e_pad = NS * k_chunks * C
    pad = e_pad - e
    u_req = num_user + (1 if pad else 0)
    u_pad = -(-u_req // (NS * C)) * (NS * C)

    ei = edge_item.astype(jnp.int32)
    eu = edge_user.astype(jnp.int32)
    if pad:
        ei = jnp.concatenate([ei, jnp.zeros((pad,), jnp.int32)])
        eu = jnp.concatenate([eu, jnp.full((pad,), num_user, jnp.int32)])
    ei3 = ei.reshape(NS, k_chunks, C)
    eu3 = eu.reshape(NS, k_chunks, C)
    ei4 = jnp.stack([2 * ei3, 2 * ei3 + 1])
    item2v = item_embedding.reshape(2 * num_item, dh)

    psums, pcnts = _sc_accumulate(ei4, eu3, item2v, u_pad)

    blk = num_user
    grid = num_user // blk
    neighbor_feat = pl.pallas_call(
        _combine_kernel,
        grid=(grid,),
        in_specs=[
            pl.BlockSpec((blk, dh), lambda i: (i, 0)),
            pl.BlockSpec((blk, dh), lambda i: (i, 0)),
            pl.BlockSpec((blk, CW), lambda i: (i, 0)),
            pl.BlockSpec((blk, CW), lambda i: (i, 0)),
        ],
        out_specs=pl.BlockSpec((blk, d), lambda i: (i, 0)),
        out_shape=jax.ShapeDtypeStruct((num_user, d), jnp.float32),
    )(psums[0, :num_user], psums[1, :num_user],
      pcnts[0, :num_user], pcnts[1, :num_user])

    return (user_embedding, neighbor_feat)


# speedup vs baseline: 1.0014x; 1.0014x over previous
import functools

import jax
import jax.numpy as jnp
from jax import lax
from jax.experimental import pallas as pl
from jax.experimental.pallas import tpu as pltpu
from jax.experimental.pallas import tpu_sc as plsc

NC = 2
NS = 16
C = 125
CW = 8
NBUF = 5


def _sc_accumulate(ei4, eu3, item2v, u_pad):
    k_chunks = eu3.shape[1]
    dh = item2v.shape[1]
    rows_per_tile = u_pad // NS
    zchunks = rows_per_tile // C

    zeros_h = jnp.zeros((C, dh), jnp.float32)
    zeros_c = jnp.zeros((C, CW), jnp.float32)
    ones_c = jnp.ones((C, CW), jnp.float32)

    mesh = plsc.VectorSubcoreMesh(core_axis_name="c", subcore_axis_name="s")

    @functools.partial(
        pl.kernel,
        mesh=mesh,
        compiler_params=pltpu.CompilerParams(use_tc_tiling_on_sc=False),
        out_type=[
            jax.ShapeDtypeStruct((NC, u_pad, dh), jnp.float32),
            jax.ShapeDtypeStruct((NC, u_pad, CW), jnp.float32),
        ],
        scratch_types=[
            pltpu.VMEM((k_chunks, C), jnp.int32),
            pltpu.VMEM((k_chunks, C), jnp.int32),
            [pltpu.VMEM((C, dh), jnp.float32)] * NBUF,
            pltpu.VMEM((C, CW), jnp.float32),
            pltpu.VMEM((C, CW), jnp.float32),
            pltpu.VMEM_SHARED((u_pad, dh), jnp.float32),
            pltpu.VMEM_SHARED((u_pad, CW), jnp.float32),
            [pltpu.SemaphoreType.DMA] * NBUF,
            [pltpu.SemaphoreType.DMA] * NBUF,
            pltpu.SemaphoreType.DMA,
        ],
    )
    def acc(ei_hbm, eu_hbm, item_hbm, z_hbm, zc_hbm, o_hbm, psums_hbm,
            pcnts_hbm, ei_v, eu_v, rows, zbuf8, ones_v, sums_sh,
            cnts_sh, gsem, ssem, csem):
        c = lax.axis_index("c")
        s = lax.axis_index("s")

        pltpu.sync_copy(z_hbm, rows[0])
        pltpu.sync_copy(zc_hbm, zbuf8)
        pltpu.sync_copy(o_hbm, ones_v)
        base = s * rows_per_tile
        for k in range(zchunks):
            pltpu.sync_copy(rows[0], sums_sh.at[pl.ds(base + k * C, C)])
            pltpu.sync_copy(zbuf8, cnts_sh.at[pl.ds(base + k * C, C)])
        pltpu.sync_copy(ei_hbm.at[c, s], ei_v)
        pltpu.sync_copy(eu_hbm.at[s], eu_v)
        plsc.subcore_barrier()

        def chunk_body(t, carry):
            j0 = t * NBUF
            gd = [pltpu.async_copy(item_hbm.at[ei_v.at[j0 + b]], rows[b],
                                   gsem[b])
                  for b in range(NBUF)]
            sd, cd = [], []
            for b in range(NBUF):
                gd[b].wait()
                sd.append(pltpu.async_copy(rows[b],
                                           sums_sh.at[eu_v.at[j0 + b]],
                                           ssem[b], add=True))
                cd.append(pltpu.async_copy(ones_v,
                                           cnts_sh.at[eu_v.at[j0 + b]],
                                           csem, add=True))
            for b in range(NBUF):
                sd[b].wait()
                cd[b].wait()
            return carry

        lax.fori_loop(0, k_chunks // NBUF, chunk_body, 0)
        plsc.subcore_barrier()

        pltpu.sync_copy(sums_sh.at[pl.ds(base, rows_per_tile)],
                        psums_hbm.at[c, pl.ds(base, rows_per_tile)])
        pltpu.sync_copy(cnts_sh.at[pl.ds(base, rows_per_tile)],
                        pcnts_hbm.at[c, pl.ds(base, rows_per_tile)])

    return acc(ei4, eu3, item2v, zeros_h, zeros_c, ones_c)


def _combine_kernel(p0, p1, c0, c1, out):
    denom = jnp.maximum((c0[:, 0:1] + c1[:, 0:1]) * 0.5, 1.0)
    out[:] = jnp.concatenate([p0[:], p1[:]], axis=1) / denom


def kernel(user_embedding, item_embedding, edge_user, edge_item):
    num_user, d = user_embedding.shape
    num_item = item_embedding.shape[0]
    dh = d // 2
    e = edge_user.shape[0]

    per_w = -(-e // NS)
    k_chunks = -(-per_w // C)
    k_chunks = -(-k_chunks // NBUF) * NBUF
    e_pad = NS * k_chunks * C
    pad = e_pad - e
    u_req = num_user + (1 if pad else 0)
    u_pad = -(-u_req // (NS * C)) * (NS * C)

    ei = edge_item.astype(jnp.int32)
    eu = edge_user.astype(jnp.int32)
    if pad:
        ei = jnp.concatenate([ei, jnp.zeros((pad,), jnp.int32)])
        eu = jnp.concatenate([eu, jnp.full((pad,), num_user, jnp.int32)])
    ei3 = ei.reshape(NS, k_chunks, C)
    eu3 = eu.reshape(NS, k_chunks, C)
    ei4 = jnp.stack([2 * ei3, 2 * ei3 + 1])
    item2v = item_embedding.reshape(2 * num_item, dh)

    psums, pcnts = _sc_accumulate(ei4, eu3, item2v, u_pad)

    blk = 2000
    grid = num_user // blk
    neighbor_feat = pl.pallas_call(
        _combine_kernel,
        grid=(grid,),
        in_specs=[
            pl.BlockSpec((blk, dh), lambda i: (i, 0)),
            pl.BlockSpec((blk, dh), lambda i: (i, 0)),
            pl.BlockSpec((blk, CW), lambda i: (i, 0)),
            pl.BlockSpec((blk, CW), lambda i: (i, 0)),
        ],
        out_specs=pl.BlockSpec((blk, d), lambda i: (i, 0)),
        out_shape=jax.ShapeDtypeStruct((num_user, d), jnp.float32),
    )(psums[0, :num_user], psums[1, :num_user],
      pcnts[0, :num_user], pcnts[1, :num_user])

    return (user_embedding, neighbor_feat)
